# baseline (speedup 1.0000x reference)
import functools

import jax
import jax.numpy as jnp
from jax import lax
from jax.experimental import pallas as pl
from jax.experimental.pallas import tpu as pltpu
from jax.experimental.pallas import tpu_sc as plsc

N = 10000
E = 320000
IN_DIM = 128
HID = 64
OUT = 16
NUM_GRAPHS = 64

NC = 2
NS = 16
NW = NC * NS
NPAD = 10240
ROWS_PER = NPAD // NS
C = 128
KW = 80
NCHUNKS = NW * KW
EPAD = NCHUNKS * C

_mesh = plsc.VectorSubcoreMesh(core_axis_name="c", subcore_axis_name="s")
_sc_params = pltpu.CompilerParams(use_tc_tiling_on_sc=False)


HW = 16


def _make_hist():
    @functools.partial(
        pl.kernel,
        out_type=jax.ShapeDtypeStruct((NC, NPAD, HW), jnp.float32),
        mesh=_mesh,
        compiler_params=_sc_params,
        scratch_types=[
            pltpu.VMEM((KW, C), jnp.int32),
            pltpu.VMEM((C, HW), jnp.float32),
            pltpu.VMEM_SHARED((NPAD, HW), jnp.float32),
        ],
    )
    def hist(dst3_hbm, ones_hbm, zeros_hbm, out_hbm, dst_v, ones_v, acc_sh):
        c = lax.axis_index("c")
        s = lax.axis_index("s")
        wid = s * NC + c
        pltpu.sync_copy(zeros_hbm,
                        acc_sh.at[pl.ds(s * ROWS_PER, ROWS_PER)])
        pltpu.sync_copy(dst3_hbm.at[wid], dst_v)
        pltpu.sync_copy(ones_hbm, ones_v)
        plsc.subcore_barrier()

        def body(j, carry):
            pltpu.sync_copy(ones_v, acc_sh.at[dst_v.at[j]], add=True)
            return carry

        lax.fori_loop(0, KW, body, 0)
        plsc.subcore_barrier()
        pltpu.sync_copy(acc_sh.at[pl.ds(s * ROWS_PER, ROWS_PER)],
                        out_hbm.at[c].at[pl.ds(s * ROWS_PER, ROWS_PER)])

    return hist


def _make_prop(F, k_fast, k_slow):
    assert 16 * (k_fast + k_slow) == NCHUNKS
    kmax = max(k_fast, k_slow)
    nparts = 1 if k_slow == 0 else NC

    @functools.partial(
        pl.kernel,
        out_type=jax.ShapeDtypeStruct((nparts, NPAD, F), jnp.float32),
        mesh=_mesh,
        compiler_params=_sc_params,
        scratch_types=[
            pltpu.VMEM((kmax, C), jnp.int32),
            pltpu.VMEM((kmax, C), jnp.int32),
            pltpu.VMEM((C, F), jnp.float32),
            pltpu.VMEM((C, F), jnp.float32),
            pltpu.VMEM_SHARED((NPAD, F), jnp.float32),
            pltpu.SemaphoreType.DMA,
            pltpu.SemaphoreType.DMA,
        ],
    )
    def prop(g_hbm, src_hbm, dst_hbm, zeros_hbm, out_hbm,
             src_v, dst_v, r0, r1, acc_sh, s0, s1):
        rows = [r0, r1]
        sems = [s0, s1]
        c = lax.axis_index("c")
        s = lax.axis_index("s")

        @pl.when(c < nparts)
        def _():
            pltpu.sync_copy(zeros_hbm,
                            acc_sh.at[pl.ds(s * ROWS_PER, ROWS_PER)])

        plsc.subcore_barrier()

        def run(base, k):
            assert k % 2 == 0
            pltpu.sync_copy(src_hbm.at[pl.ds(base, k)], src_v.at[pl.ds(0, k)])
            pltpu.sync_copy(dst_hbm.at[pl.ds(base, k)], dst_v.at[pl.ds(0, k)])
            pltpu.async_copy(g_hbm.at[src_v.at[0]], rows[0], sems[0])

            def body(i, carry):
                for b in range(2):
                    j = 2 * i + b
                    pltpu.make_async_copy(g_hbm.at[src_v.at[j]], rows[b],
                                          sems[b]).wait()

                    @pl.when(j + 1 < k)
                    def _():
                        nb = 1 - b
                        pltpu.async_copy(g_hbm.at[src_v.at[j + 1]],
                                         rows[nb], sems[nb])

                    pltpu.sync_copy(rows[b], acc_sh.at[dst_v.at[j]], add=True)
                return carry

            lax.fori_loop(0, k // 2, body, 0)

        @pl.when(c == 0)
        def _():
            run(s * k_fast, k_fast)

        if k_slow > 0:
            @pl.when(c == 1)
            def _():
                run(16 * k_fast + s * k_slow, k_slow)

        plsc.subcore_barrier()

        @pl.when(c < nparts)
        def _():
            pltpu.sync_copy(acc_sh.at[pl.ds(s * ROWS_PER, ROWS_PER)],
                            out_hbm.at[c].at[pl.ds(s * ROWS_PER, ROWS_PER)])

    return prop


_hist = _make_hist()
_prop64 = _make_prop(HID, 144, 16)
_prop16 = _make_prop(OUT, 112, 48)


_RB = 1280
_GRID = NPAD // _RB


def _stage1_body(x_ref, w_ref, hist_ref, g_ref, dis_ref):
    pid = pl.program_id(0)
    deg = hist_ref[0][:, 0:1] + hist_ref[1][:, 0:1] + 1.0
    row = pid * _RB + lax.broadcasted_iota(jnp.int32, (_RB, 1), 0)
    dis = jnp.where(row < N, lax.rsqrt(deg), 0.0)
    h = jnp.dot(x_ref[...], w_ref[...], preferred_element_type=jnp.float32)
    g_ref[...] = h * dis
    dis_ref[...] = dis


def _tc_stage1(x_pad, W1, hist):
    return pl.pallas_call(
        _stage1_body,
        grid=(_GRID,),
        in_specs=[
            pl.BlockSpec((_RB, IN_DIM), lambda i: (i, 0)),
            pl.BlockSpec((IN_DIM, HID), lambda i: (0, 0)),
            pl.BlockSpec((NC, _RB, HW), lambda i: (0, i, 0)),
        ],
        out_specs=[
            pl.BlockSpec((_RB, HID), lambda i: (i, 0)),
            pl.BlockSpec((_RB, 1), lambda i: (i, 0)),
        ],
        out_shape=[
            jax.ShapeDtypeStruct((NPAD, HID), jnp.float32),
            jax.ShapeDtypeStruct((NPAD, 1), jnp.float32),
        ],
    )(x_pad, W1, hist)


def _mid_body(acc_ref, g_ref, dis_ref, b_ref, w_ref, out_ref):
    dis = dis_ref[...]
    asum = acc_ref[0]
    for p in range(1, acc_ref.shape[0]):
        asum = asum + acc_ref[p]
    z = dis * (asum + g_ref[...]) + b_ref[...]
    z = jnp.maximum(z, 0.0)
    out_ref[...] = jnp.dot(z, w_ref[...], preferred_element_type=jnp.float32) * dis


def _tc_mid(acc, g, dis, b2d, W, fout):
    fin = g.shape[1]
    nparts = acc.shape[0]
    return pl.pallas_call(
        _mid_body,
        grid=(_GRID,),
        in_specs=[
            pl.BlockSpec((nparts, _RB, fin), lambda i: (0, i, 0)),
            pl.BlockSpec((_RB, fin), lambda i: (i, 0)),
            pl.BlockSpec((_RB, 1), lambda i: (i, 0)),
            pl.BlockSpec((1, fin), lambda i: (0, 0)),
            pl.BlockSpec((fin, fout), lambda i: (0, 0)),
        ],
        out_specs=pl.BlockSpec((_RB, fout), lambda i: (i, 0)),
        out_shape=jax.ShapeDtypeStruct((NPAD, fout), jnp.float32),
    )(acc, g, dis, b2d, W)


def _final_body(acc_ref, g_ref, dis_ref, b_ref, batch_ref, out_ref):
    asum = acc_ref[0]
    for p in range(1, acc_ref.shape[0]):
        asum = asum + acc_ref[p]
    node = dis_ref[...] * (asum + g_ref[...]) + b_ref[...]
    gids = lax.broadcasted_iota(jnp.int32, (NUM_GRAPHS, NPAD), 0)
    mask = (batch_ref[...] == gids).astype(jnp.float32)
    sums = jnp.dot(mask, node, preferred_element_type=jnp.float32)
    cnts = jnp.sum(mask, axis=1, keepdims=True)
    out_ref[...] = sums / jnp.maximum(cnts, 1.0)


def _tc_final(acc, g3, dis, b3_2d, batch2d):
    return pl.pallas_call(
        _final_body,
        out_shape=jax.ShapeDtypeStruct((NUM_GRAPHS, OUT), jnp.float32),
    )(acc, g3, dis, b3_2d, batch2d)


def kernel(x, edge_index, batch, W1, b1, W2, b2, W3, b3):
    src = jnp.concatenate([edge_index[0],
                           jnp.full((EPAD - E,), N, dtype=jnp.int32)])
    dst = jnp.concatenate([edge_index[1],
                           jnp.full((EPAD - E,), N, dtype=jnp.int32)])
    src3 = src.reshape(NCHUNKS, C)
    dst3 = dst.reshape(NCHUNKS, C)

    x_pad = jnp.pad(x, ((0, NPAD - N), (0, 0)))
    batch2d = jnp.pad(batch, (0, NPAD - N),
                      constant_values=NUM_GRAPHS).reshape(1, NPAD)
    zeros64 = jnp.zeros((ROWS_PER, HID), jnp.float32)
    zeros16 = jnp.zeros((ROWS_PER, OUT), jnp.float32)
    zeros1 = jnp.zeros((ROWS_PER, HW), jnp.float32)
    ones_c = jnp.ones((C, HW), jnp.float32)

    hist = _hist(dst3.reshape(NW, KW, C), ones_c, zeros1)
    g1, dis = _tc_stage1(x_pad, W1, hist)

    acc1 = _prop64(g1, src3, dst3, zeros64)
    g2 = _tc_mid(acc1, g1, dis, b1.reshape(1, HID), W2, HID)

    acc2 = _prop64(g2, src3, dst3, zeros64)
    g3 = _tc_mid(acc2, g2, dis, b2.reshape(1, HID), W3, OUT)

    acc3 = _prop16(g3, src3, dst3, zeros16)
    return _tc_final(acc3, g3, dis, b3.reshape(1, OUT), batch2d)


# speedup vs baseline: 1.0809x; 1.0809x over previous
import functools

import jax
import jax.numpy as jnp
from jax import lax
from jax.experimental import pallas as pl
from jax.experimental.pallas import tpu as pltpu
from jax.experimental.pallas import tpu_sc as plsc

N = 10000
E = 320000
IN_DIM = 128
HID = 64
OUT = 16
NUM_GRAPHS = 64

NC = 2
NS = 16
NW = NC * NS
NPAD = 10240
ROWS_PER = NPAD // NS
C = 128
KW = 80
NCHUNKS = NW * KW
EPAD = NCHUNKS * C

_mesh = plsc.VectorSubcoreMesh(core_axis_name="c", subcore_axis_name="s")
_sc_params = pltpu.CompilerParams(use_tc_tiling_on_sc=False)


HW = 16


def _make_hist():
    @functools.partial(
        pl.kernel,
        out_type=jax.ShapeDtypeStruct((NC, NPAD, HW), jnp.float32),
        mesh=_mesh,
        compiler_params=_sc_params,
        scratch_types=[
            pltpu.VMEM((KW, C), jnp.int32),
            pltpu.VMEM((C, HW), jnp.float32),
            pltpu.VMEM_SHARED((NPAD, HW), jnp.float32),
        ],
    )
    def hist(dst3_hbm, ones_hbm, zeros_hbm, out_hbm, dst_v, ones_v, acc_sh):
        c = lax.axis_index("c")
        s = lax.axis_index("s")
        wid = s * NC + c
        pltpu.sync_copy(zeros_hbm,
                        acc_sh.at[pl.ds(s * ROWS_PER, ROWS_PER)])
        pltpu.sync_copy(dst3_hbm.at[wid], dst_v)
        pltpu.sync_copy(ones_hbm, ones_v)
        plsc.subcore_barrier()

        def body(j, carry):
            pltpu.sync_copy(ones_v, acc_sh.at[dst_v.at[j]], add=True)
            return carry

        lax.fori_loop(0, KW, body, 0)
        plsc.subcore_barrier()
        pltpu.sync_copy(acc_sh.at[pl.ds(s * ROWS_PER, ROWS_PER)],
                        out_hbm.at[c].at[pl.ds(s * ROWS_PER, ROWS_PER)])

    return hist


def _make_prop(F, k_fast, k_slow):
    assert 16 * (k_fast + k_slow) == NCHUNKS
    kmax = max(k_fast, k_slow)
    nparts = 1 if k_slow == 0 else NC

    @functools.partial(
        pl.kernel,
        out_type=jax.ShapeDtypeStruct((nparts, NPAD, F), jnp.float32),
        mesh=_mesh,
        compiler_params=_sc_params,
        scratch_types=[
            pltpu.VMEM((kmax, C), jnp.int32),
            pltpu.VMEM((kmax, C), jnp.int32),
            pltpu.VMEM((C, F), jnp.float32),
            pltpu.VMEM((C, F), jnp.float32),
            pltpu.VMEM_SHARED((NPAD, F), jnp.float32),
            pltpu.SemaphoreType.DMA,
            pltpu.SemaphoreType.DMA,
        ],
    )
    def prop(g_hbm, src_hbm, dst_hbm, zeros_hbm, out_hbm,
             src_v, dst_v, r0, r1, acc_sh, s0, s1):
        rows = [r0, r1]
        sems = [s0, s1]
        c = lax.axis_index("c")
        s = lax.axis_index("s")

        @pl.when(c < nparts)
        def _():
            pltpu.sync_copy(zeros_hbm,
                            acc_sh.at[pl.ds(s * ROWS_PER, ROWS_PER)])

        plsc.subcore_barrier()

        def run(base, k):
            assert k % 2 == 0
            pltpu.sync_copy(src_hbm.at[pl.ds(base, k)], src_v.at[pl.ds(0, k)])
            pltpu.sync_copy(dst_hbm.at[pl.ds(base, k)], dst_v.at[pl.ds(0, k)])
            pltpu.async_copy(g_hbm.at[src_v.at[0]], rows[0], sems[0])

            def body(i, carry):
                for b in range(2):
                    j = 2 * i + b
                    pltpu.make_async_copy(g_hbm.at[src_v.at[j]], rows[b],
                                          sems[b]).wait()

                    @pl.when(j + 1 < k)
                    def _():
                        nb = 1 - b
                        pltpu.async_copy(g_hbm.at[src_v.at[j + 1]],
                                         rows[nb], sems[nb])

                    pltpu.sync_copy(rows[b], acc_sh.at[dst_v.at[j]], add=True)
                return carry

            lax.fori_loop(0, k // 2, body, 0)

        @pl.when(c == 0)
        def _():
            run(s * k_fast, k_fast)

        if k_slow > 0:
            @pl.when(c == 1)
            def _():
                run(16 * k_fast + s * k_slow, k_slow)

        plsc.subcore_barrier()

        @pl.when(c < nparts)
        def _():
            pltpu.sync_copy(acc_sh.at[pl.ds(s * ROWS_PER, ROWS_PER)],
                            out_hbm.at[c].at[pl.ds(s * ROWS_PER, ROWS_PER)])

    return prop


_hist = _make_hist()
_prop64 = _make_prop(HID, 136, 24)
_prop16 = _make_prop(OUT, 104, 56)


_RB = 1280
_GRID = NPAD // _RB


def _stage1_body(x_ref, w_ref, hist_ref, g_ref, dis_ref):
    pid = pl.program_id(0)
    deg = hist_ref[0][:, 0:1] + hist_ref[1][:, 0:1] + 1.0
    row = pid * _RB + lax.broadcasted_iota(jnp.int32, (_RB, 1), 0)
    dis = jnp.where(row < N, lax.rsqrt(deg), 0.0)
    h = jnp.dot(x_ref[...], w_ref[...], preferred_element_type=jnp.float32)
    g_ref[...] = h * dis
    dis_ref[...] = dis


def _tc_stage1(x_pad, W1, hist):
    return pl.pallas_call(
        _stage1_body,
        grid=(_GRID,),
        in_specs=[
            pl.BlockSpec((_RB, IN_DIM), lambda i: (i, 0)),
            pl.BlockSpec((IN_DIM, HID), lambda i: (0, 0)),
            pl.BlockSpec((NC, _RB, HW), lambda i: (0, i, 0)),
        ],
        out_specs=[
            pl.BlockSpec((_RB, HID), lambda i: (i, 0)),
            pl.BlockSpec((_RB, 1), lambda i: (i, 0)),
        ],
        out_shape=[
            jax.ShapeDtypeStruct((NPAD, HID), jnp.float32),
            jax.ShapeDtypeStruct((NPAD, 1), jnp.float32),
        ],
    )(x_pad, W1, hist)


def _mid_body(acc_ref, g_ref, dis_ref, b_ref, w_ref, out_ref):
    dis = dis_ref[...]
    asum = acc_ref[0]
    for p in range(1, acc_ref.shape[0]):
        asum = asum + acc_ref[p]
    z = dis * (asum + g_ref[...]) + b_ref[...]
    z = jnp.maximum(z, 0.0)
    out_ref[...] = jnp.dot(z, w_ref[...], preferred_element_type=jnp.float32) * dis


def _tc_mid(acc, g, dis, b2d, W, fout):
    fin = g.shape[1]
    nparts = acc.shape[0]
    return pl.pallas_call(
        _mid_body,
        grid=(_GRID,),
        in_specs=[
            pl.BlockSpec((nparts, _RB, fin), lambda i: (0, i, 0)),
            pl.BlockSpec((_RB, fin), lambda i: (i, 0)),
            pl.BlockSpec((_RB, 1), lambda i: (i, 0)),
            pl.BlockSpec((1, fin), lambda i: (0, 0)),
            pl.BlockSpec((fin, fout), lambda i: (0, 0)),
        ],
        out_specs=pl.BlockSpec((_RB, fout), lambda i: (i, 0)),
        out_shape=jax.ShapeDtypeStruct((NPAD, fout), jnp.float32),
    )(acc, g, dis, b2d, W)


def _final_body(acc_ref, g_ref, dis_ref, b_ref, batch_ref, out_ref):
    asum = acc_ref[0]
    for p in range(1, acc_ref.shape[0]):
        asum = asum + acc_ref[p]
    node = dis_ref[...] * (asum + g_ref[...]) + b_ref[...]
    gids = lax.broadcasted_iota(jnp.int32, (NUM_GRAPHS, NPAD), 0)
    mask = (batch_ref[...] == gids).astype(jnp.float32)
    sums = jnp.dot(mask, node, preferred_element_type=jnp.float32)
    cnts = jnp.sum(mask, axis=1, keepdims=True)
    out_ref[...] = sums / jnp.maximum(cnts, 1.0)


def _tc_final(acc, g3, dis, b3_2d, batch2d):
    return pl.pallas_call(
        _final_body,
        out_shape=jax.ShapeDtypeStruct((NUM_GRAPHS, OUT), jnp.float32),
    )(acc, g3, dis, b3_2d, batch2d)


def kernel(x, edge_index, batch, W1, b1, W2, b2, W3, b3):
    src = jnp.concatenate([edge_index[0],
                           jnp.full((EPAD - E,), N, dtype=jnp.int32)])
    dst = jnp.concatenate([edge_index[1],
                           jnp.full((EPAD - E,), N, dtype=jnp.int32)])
    src3 = src.reshape(NCHUNKS, C)
    dst3 = dst.reshape(NCHUNKS, C)

    x_pad = jnp.pad(x, ((0, NPAD - N), (0, 0)))
    batch2d = jnp.pad(batch, (0, NPAD - N),
                      constant_values=NUM_GRAPHS).reshape(1, NPAD)
    zeros64 = jnp.zeros((ROWS_PER, HID), jnp.float32)
    zeros16 = jnp.zeros((ROWS_PER, OUT), jnp.float32)
    zeros1 = jnp.zeros((ROWS_PER, HW), jnp.float32)
    ones_c = jnp.ones((C, HW), jnp.float32)

    hist = _hist(dst3.reshape(NW, KW, C), ones_c, zeros1)
    g1, dis = _tc_stage1(x_pad, W1, hist)

    acc1 = _prop64(g1, src3, dst3, zeros64)
    g2 = _tc_mid(acc1, g1, dis, b1.reshape(1, HID), W2, HID)

    acc2 = _prop64(g2, src3, dst3, zeros64)
    g3 = _tc_mid(acc2, g2, dis, b2.reshape(1, HID), W3, OUT)

    acc3 = _prop16(g3, src3, dst3, zeros16)
    return _tc_final(acc3, g3, dis, b3.reshape(1, OUT), batch2d)
